# Initial kernel scaffold; baseline (speedup 1.0000x reference)
import jax
import jax.numpy as jnp
from jax.experimental import pallas as pl


def kernel(x, table):
    raise NotImplementedError("write your pallas kernel here")


# speedup vs baseline: 1.8685x; 1.8685x over previous
import functools

import jax
import jax.numpy as jnp
from jax import lax
from jax.experimental import pallas as pl
from jax.experimental.pallas import tpu as pltpu
from jax.experimental.pallas import tpu_sc as plsc

B, S = 1024, 200
VOCAB, C, D = 100000, 4, 64
CD = C * D
WIN = 104
OUT_PER_TASK = 100
TASKS = B * 2
NC, NS = 2, 16
NW = NC * NS
TASKS_PER_W = TASKS // NW


def _sc_body(x2win_hbm, table_hbm, out_hbm, idx_all, emb, outb):
    wid = lax.axis_index("s") * NC + lax.axis_index("c")
    base_task = wid * TASKS_PER_W
    pltpu.sync_copy(x2win_hbm.at[pl.ds(base_task, TASKS_PER_W)], idx_all)

    def task_body(t, _):
        pltpu.sync_copy(table_hbm.at[idx_all.at[t]], emb)

        def s_body(s, _):
            for g in range(D // 16):
                acc = emb[s, pl.ds(g * 16, 16)]
                for i in range(1, C):
                    acc = acc + emb[s + i, pl.ds(i * D + g * 16, 16)]
                outb[s, pl.ds(g * 16, 16)] = acc
            return 0

        lax.fori_loop(0, OUT_PER_TASK, s_body, 0)
        pltpu.sync_copy(outb, out_hbm.at[base_task + t])
        return 0

    lax.fori_loop(0, TASKS_PER_W, task_body, 0)


def kernel(x, table):
    x = x.astype(jnp.int32)
    x2 = jnp.concatenate([x, jnp.zeros((B, C), jnp.int32)], axis=1)
    x2win = jnp.stack([x2[:, :WIN], x2[:, S - OUT_PER_TASK:]], axis=1)
    x2win = x2win.reshape(TASKS, WIN)
    table2d = table.reshape(VOCAB, CD)

    mesh = plsc.VectorSubcoreMesh(core_axis_name="c", subcore_axis_name="s")
    run = functools.partial(
        pl.kernel,
        mesh=mesh,
        out_type=jax.ShapeDtypeStruct((TASKS, OUT_PER_TASK, D), jnp.float32),
        scratch_types=[
            pltpu.VMEM((TASKS_PER_W, WIN), jnp.int32),
            pltpu.VMEM((WIN, CD), jnp.float32),
            pltpu.VMEM((OUT_PER_TASK, D), jnp.float32),
        ],
    )(_sc_body)
    out = run(x2win, table2d)
    return out.reshape(B, S, D)


# speedup vs baseline: 1.9454x; 1.0412x over previous
import functools

import jax
import jax.numpy as jnp
from jax import lax
from jax.experimental import pallas as pl
from jax.experimental.pallas import tpu as pltpu
from jax.experimental.pallas import tpu_sc as plsc

B, S = 1024, 200
VOCAB, C, D = 100000, 4, 64
CD = C * D
WIN = 104
OUT_PER_TASK = 100
TASKS = B * 2
NC, NS = 2, 16
NW = NC * NS
TASKS_PER_W = TASKS // NW


def _compute(emb, outb):
    def s_body(s, _):
        for g in range(D // 16):
            acc = emb[s, pl.ds(g * 16, 16)]
            for i in range(1, C):
                acc = acc + emb[s + i, pl.ds(i * D + g * 16, 16)]
            outb[s, pl.ds(g * 16, 16)] = acc
        return 0

    lax.fori_loop(0, OUT_PER_TASK, s_body, 0, unroll=2)


def _sc_body(x2win_hbm, table_hbm, out_hbm, idx_all,
             emb0, emb1, out0, out1, sg0, sg1, so0, so1):
    wid = lax.axis_index("s") * NC + lax.axis_index("c")
    base_task = wid * TASKS_PER_W
    pltpu.sync_copy(x2win_hbm.at[pl.ds(base_task, TASKS_PER_W)], idx_all)

    def gather(t, embb, sem):
        return pltpu.async_copy(table_hbm.at[idx_all.at[t]], embb, sem)

    def gather_wait(t, embb, sem):
        pltpu.make_async_copy(table_hbm.at[idx_all.at[t]], embb, sem).wait()

    def scatter(t, outb, sem):
        return pltpu.async_copy(outb, out_hbm.at[base_task + t], sem)

    def scatter_wait(t, outb, sem):
        pltpu.make_async_copy(outb, out_hbm.at[base_task + t], sem).wait()

    gather(0, emb0, sg0)

    def task_body(k, _):
        t0 = 2 * k
        gather(t0 + 1, emb1, sg1)
        gather_wait(t0, emb0, sg0)

        @pl.when(k >= 1)
        def _():
            scatter_wait(t0 - 2, out0, so0)

        _compute(emb0, out0)
        scatter(t0, out0, so0)

        @pl.when(k < TASKS_PER_W // 2 - 1)
        def _():
            gather(t0 + 2, emb0, sg0)

        gather_wait(t0 + 1, emb1, sg1)

        @pl.when(k >= 1)
        def _():
            scatter_wait(t0 - 1, out1, so1)

        _compute(emb1, out1)
        scatter(t0 + 1, out1, so1)
        return 0

    lax.fori_loop(0, TASKS_PER_W // 2, task_body, 0)
    scatter_wait(TASKS_PER_W - 2, out0, so0)
    scatter_wait(TASKS_PER_W - 1, out1, so1)


def kernel(x, table):
    x = x.astype(jnp.int32)
    x2 = jnp.concatenate([x, jnp.zeros((B, C), jnp.int32)], axis=1)
    x2win = jnp.stack([x2[:, :WIN], x2[:, S - OUT_PER_TASK:]], axis=1)
    x2win = x2win.reshape(TASKS, WIN)
    table2d = table.reshape(VOCAB, CD)

    mesh = plsc.VectorSubcoreMesh(core_axis_name="c", subcore_axis_name="s")
    run = functools.partial(
        pl.kernel,
        mesh=mesh,
        out_type=jax.ShapeDtypeStruct((TASKS, OUT_PER_TASK, D), jnp.float32),
        scratch_types=[
            pltpu.VMEM((TASKS_PER_W, WIN), jnp.int32),
            pltpu.VMEM((WIN, CD), jnp.float32),
            pltpu.VMEM((WIN, CD), jnp.float32),
            pltpu.VMEM((OUT_PER_TASK, D), jnp.float32),
            pltpu.VMEM((OUT_PER_TASK, D), jnp.float32),
            pltpu.SemaphoreType.DMA,
            pltpu.SemaphoreType.DMA,
            pltpu.SemaphoreType.DMA,
            pltpu.SemaphoreType.DMA,
        ],
    )(_sc_body)
    out = run(x2win, table2d)
    return out.reshape(B, S, D)
